# Initial kernel scaffold; baseline (speedup 1.0000x reference)
import jax
import jax.numpy as jnp
from jax.experimental import pallas as pl


def kernel(pred, target):
    raise NotImplementedError("write your pallas kernel here")


# speedup vs baseline: 17219.6153x; 17219.6153x over previous
import functools

import jax
import jax.numpy as jnp
from jax.experimental import pallas as pl


def _emd_reduce_kernel(p_ref, t_ref, o_ref, *, inv_n, inv_b):
    p = p_ref[:]
    t = t_ref[:]
    total = jnp.sum(p * p + t * t, keepdims=True)
    sp = jnp.sum(p, axis=1, keepdims=True)
    st = jnp.sum(t, axis=1, keepdims=True)
    cross = jnp.sum(sp * st, keepdims=True)
    o_ref[:, :] = (total - 2.0 * inv_n * cross) * inv_b


def kernel(pred, target):
    b, n, c = pred.shape
    p = pred.transpose(0, 2, 1).reshape(b * c, n)
    t = target.transpose(0, 2, 1).reshape(b * c, n)
    out = pl.pallas_call(
        functools.partial(_emd_reduce_kernel, inv_n=1.0 / n, inv_b=1.0 / b),
        out_shape=jax.ShapeDtypeStruct((1, 1), jnp.float32),
    )(p, t)
    return out[0, 0]
